# Initial kernel scaffold; baseline (speedup 1.0000x reference)
import functools

import jax
import jax.numpy as jnp
from jax.experimental import pallas as pl
from jax.experimental.pallas import tpu as pltpu
from jax.experimental.pallas import tpu_sc as plsc

B = 1024
D_IN = 768
H = 512
E = 768
K = 8
N = 100000
OUT = 128

NB = 2048
NPAD = 100352
NBLK = NPAD // NB
CHUNKS = NB // 128

_NEG_INF = float("-inf")


def _mlp2(x, w1, b1, w2, b2):
    return jax.nn.relu(x @ w1 + b1) @ w2 + b2


def _normalize(x, eps=1e-12):
    n = jnp.linalg.norm(x, axis=1, keepdims=True)
    return x / jnp.maximum(n, eps)


def _insert_candidate(x, xi, v1, v2, v3, v4, c1, c2, c3, c4):
    r1, r2, r3, r4 = v1[...], v2[...], v3[...], v4[...]
    q1, q2, q3, q4 = c1[...], c2[...], c3[...], c4[...]
    g1 = x > r1
    g2 = x > r2
    g3 = x > r3
    g4 = x > r4
    v4[...] = jnp.where(g3, r3, jnp.where(g4, x, r4))
    c4[...] = jnp.where(g3, q3, jnp.where(g4, xi, q4))
    v3[...] = jnp.where(g2, r2, jnp.where(g3, x, r3))
    c3[...] = jnp.where(g2, q2, jnp.where(g3, xi, q3))
    v2[...] = jnp.where(g1, r1, jnp.where(g2, x, r2))
    c2[...] = jnp.where(g1, q1, jnp.where(g2, xi, q2))
    v1[...] = jnp.where(g1, x, r1)
    c1[...] = jnp.where(g1, xi, q1)


def _score_body(qb_ref, corpn_ref, corpraw_ref, nuw1_ref, nub1_ref, nuw2_ref,
                nub2_ref, scores_ref, idx_ref,
                v1, v2, v3, v4, c1, c2, c3, c4):
    i = pl.program_id(0)

    @pl.when(i == 0)
    def _init():
        neg = jnp.full((B, 128), _NEG_INF, jnp.float32)
        zero = jnp.zeros((B, 128), jnp.int32)
        v1[...] = neg
        v2[...] = neg
        v3[...] = neg
        v4[...] = neg
        c1[...] = zero
        c2[...] = zero
        c3[...] = zero
        c4[...] = zero

    h = jax.lax.dot_general(
        corpraw_ref[...], nuw1_ref[...],
        (((1,), (0,)), ((), ())), preferred_element_type=jnp.float32)
    h = jnp.maximum(h + nub1_ref[...], 0.0)
    nu2 = jax.lax.dot_general(
        nuw2_ref[...], h.astype(jnp.bfloat16),
        (((1,), (1,)), ((), ())), preferred_element_type=jnp.float32)
    nu = jax.nn.sigmoid(nu2 + nub2_ref[...])

    sim = jax.lax.dot_general(
        qb_ref[...], corpn_ref[...],
        (((1,), (1,)), ((), ())), preferred_element_type=jnp.float32)
    w = sim * nu

    m1 = jnp.full((B, 128), _NEG_INF, jnp.float32)
    m2 = jnp.full((B, 128), _NEG_INF, jnp.float32)
    j1 = jnp.zeros((B, 128), jnp.int32)
    j2 = jnp.zeros((B, 128), jnp.int32)
    for c in range(CHUNKS):
        x = w[:, c * 128:(c + 1) * 128]
        bc = i * CHUNKS + c
        g1 = x > m1
        g2 = x > m2
        j2 = jnp.where(g1, j1, jnp.where(g2, bc, j2))
        m2 = jnp.where(g1, m1, jnp.where(g2, x, m2))
        j1 = jnp.where(g1, bc, j1)
        m1 = jnp.where(g1, x, m1)

    _insert_candidate(m1, j1, v1, v2, v3, v4, c1, c2, c3, c4)
    _insert_candidate(m2, j2, v1, v2, v3, v4, c1, c2, c3, c4)

    @pl.when(i == NBLK - 1)
    def _extract():
        cand_v = jnp.concatenate([v1[...], v2[...], v3[...], v4[...]], axis=1)
        cand_b = jnp.concatenate([c1[...], c2[...], c3[...], c4[...]], axis=1)
        lane = jax.lax.broadcasted_iota(jnp.int32, (B, 512), 1) % 128
        gcol = cand_b * 128 + lane
        cand_v = jnp.where(gcol >= N, _NEG_INF, cand_v)
        lane128 = jax.lax.broadcasted_iota(jnp.int32, (B, 128), 1)
        s_out = jnp.zeros((B, 128), jnp.float32)
        i_out = jnp.zeros((B, 128), jnp.int32)
        big = jnp.int32(2**31 - 1)
        for k in range(K):
            vmax = jnp.max(cand_v, axis=1, keepdims=True)
            ism = cand_v == vmax
            selidx = jnp.min(jnp.where(ism, gcol, big), axis=1, keepdims=True)
            s_out = jnp.where(lane128 == k, vmax, s_out)
            i_out = jnp.where(lane128 == k, selidx, i_out)
            cand_v = jnp.where(ism & (gcol == selidx), _NEG_INF, cand_v)
        scores_ref[...] = s_out
        idx_ref[...] = i_out


def _score_topk(qb, corpn_b, corpraw_b, nuw1_b, nub1, nuw2_b, nub2,
                interpret=False):
    scores_p, idx_p = pl.pallas_call(
        _score_body,
        grid=(NBLK,),
        in_specs=[
            pl.BlockSpec((B, E), lambda i: (0, 0)),
            pl.BlockSpec((NB, E), lambda i: (i, 0)),
            pl.BlockSpec((NB, E), lambda i: (i, 0)),
            pl.BlockSpec((E, 128), lambda i: (0, 0)),
            pl.BlockSpec((1, 128), lambda i: (0, 0)),
            pl.BlockSpec((1, 128), lambda i: (0, 0)),
            pl.BlockSpec((1, 1), lambda i: (0, 0)),
        ],
        out_specs=[
            pl.BlockSpec((B, 128), lambda i: (0, 0)),
            pl.BlockSpec((B, 128), lambda i: (0, 0)),
        ],
        out_shape=[
            jax.ShapeDtypeStruct((B, 128), jnp.float32),
            jax.ShapeDtypeStruct((B, 128), jnp.int32),
        ],
        scratch_shapes=[pltpu.VMEM((B, 128), jnp.float32)] * 4
                      + [pltpu.VMEM((B, 128), jnp.int32)] * 4,
        compiler_params=pltpu.CompilerParams(
            dimension_semantics=("arbitrary",)),
        interpret=interpret,
    )(qb, corpn_b, corpraw_b, nuw1_b, nub1, nuw2_b, nub2)
    return scores_p[:, :K], idx_p[:, :K]


_GW = 64


def _sc_gather(corpus, indices_flat):
    mesh = plsc.VectorSubcoreMesh(core_axis_name="core",
                                  subcore_axis_name="subcore")
    num_idx = indices_flat.shape[1]

    @functools.partial(
        pl.kernel,
        out_type=jax.ShapeDtypeStruct((num_idx, E), jnp.float32),
        mesh=mesh)
    def gk(x_hbm, i_hbm, o_hbm):
        def body(i_vmem, o_vmem):
            pltpu.sync_copy(x_hbm.at[i_vmem.at[0]], o_vmem)

        pltpu.emit_pipeline(
            body,
            grid=(num_idx // _GW,),
            in_specs=[pl.BlockSpec((1, _GW), lambda i: (0, i))],
            out_specs=[pl.BlockSpec((_GW, E), lambda i: (i, 0))],
            core_axis_name=("core", "subcore"),
            dimension_semantics=(pltpu.PARALLEL,),
        )(i_hbm, o_hbm)

    return gk(corpus, indices_flat)


_HI = jax.lax.Precision.HIGHEST


def _head_body(flat_ref, t_ref, trw_ref, trb_ref, cfb_ref, rew_ref, reb_ref,
               w1_ref, b1_ref, w2_ref, b2_ref, w3_ref, b3_ref, out_ref):
    t_enc = jnp.dot(t_ref[...], trw_ref[...], precision=_HI) + trb_ref[...]
    r_enc = jnp.dot(flat_ref[...], rew_ref[...], precision=_HI) + reb_ref[...]
    cpart = jnp.dot(cfb_ref[...], w1_ref[H:2 * H, :], precision=_HI)
    h1 = jnp.maximum(
        jnp.dot(t_enc, w1_ref[0:H, :], precision=_HI) + cpart
        + jnp.dot(r_enc, w1_ref[2 * H:3 * H, :], precision=_HI)
        + b1_ref[...], 0.0)
    h2 = jnp.maximum(jnp.dot(h1, w2_ref[...], precision=_HI) + b2_ref[...], 0.0)
    out_ref[...] = jnp.dot(h2, w3_ref[...], precision=_HI) + b3_ref[...]


def _head(flat, treatment, tr_w, tr_b, cf_b, re_w, re_b,
          op_w1, op_b1, op_w2, op_b2, op_w3, op_b3, interpret=False):
    RB = 256
    return pl.pallas_call(
        _head_body,
        grid=(B // RB,),
        in_specs=[
            pl.BlockSpec((RB, K * E), lambda i: (i, 0)),
            pl.BlockSpec((RB, D_IN), lambda i: (i, 0)),
            pl.BlockSpec((D_IN, H), lambda i: (0, 0)),
            pl.BlockSpec((1, H), lambda i: (0, 0)),
            pl.BlockSpec((1, H), lambda i: (0, 0)),
            pl.BlockSpec((K * E, H), lambda i: (0, 0)),
            pl.BlockSpec((1, H), lambda i: (0, 0)),
            pl.BlockSpec((3 * H, H), lambda i: (0, 0)),
            pl.BlockSpec((1, H), lambda i: (0, 0)),
            pl.BlockSpec((H, H // 2), lambda i: (0, 0)),
            pl.BlockSpec((1, H // 2), lambda i: (0, 0)),
            pl.BlockSpec((H // 2, OUT), lambda i: (0, 0)),
            pl.BlockSpec((1, OUT), lambda i: (0, 0)),
        ],
        out_specs=pl.BlockSpec((RB, OUT), lambda i: (i, 0)),
        out_shape=jax.ShapeDtypeStruct((B, OUT), jnp.float32),
        compiler_params=pltpu.CompilerParams(
            dimension_semantics=("arbitrary",)),
        interpret=interpret,
    )(flat, treatment, tr_w, tr_b, cf_b, re_w, re_b,
      op_w1, op_b1, op_w2, op_b2, op_w3, op_b3)


def kernel(patient, treatment, confounders, outcome_features, corpus_embeddings,
           pop_w1, pop_b1, pop_w2, pop_b2, int_w1, int_b1, int_w2, int_b2,
           cmp_w1, cmp_b1, cmp_w2, cmp_b2, out_w1, out_b1, out_w2, out_b2,
           eff_w1, eff_b1, eff_w2, eff_b2, nu_w1, nu_b1, nu_w2, nu_b2,
           tr_w, tr_b, cf_w, cf_b, re_w, re_b,
           op_w1, op_b1, op_w2, op_b2, op_w3, op_b3):
    control = jnp.zeros_like(treatment)
    population = _mlp2(patient, pop_w1, pop_b1, pop_w2, pop_b2)
    intervention = _mlp2(treatment, int_w1, int_b1, int_w2, int_b2)
    comparison = _mlp2(control, cmp_w1, cmp_b1, cmp_w2, cmp_b2)
    pi = jnp.concatenate([population, intervention], axis=1)
    pc = jnp.concatenate([population, comparison], axis=1)
    pi_eff = _mlp2(pi, eff_w1, eff_b1, eff_w2, eff_b2)
    pc_eff = _mlp2(pc, eff_w1, eff_b1, eff_w2, eff_b2)
    cemb = _normalize(pi_eff - pc_eff)
    q = _normalize(cemb)
    corp = _normalize(corpus_embeddings)

    pad = NPAD - N
    qb = q.astype(jnp.bfloat16)
    corpn_b = jnp.pad(corp, ((0, pad), (0, 0))).astype(jnp.bfloat16)
    corpraw_b = jnp.pad(corpus_embeddings, ((0, pad), (0, 0))).astype(jnp.bfloat16)
    nuw1_b = nu_w1.astype(jnp.bfloat16)
    nub1 = nu_b1.reshape(1, 128)
    nuw2_b = nu_w2.reshape(1, 128).astype(jnp.bfloat16)
    nub2 = nu_b2.reshape(1, 1)

    scores, indices = _score_topk(qb, corpn_b, corpraw_b, nuw1_b, nub1,
                                  nuw2_b, nub2)

    retrieved = _sc_gather(corpus_embeddings, indices.reshape(1, B * K))
    flat = retrieved.reshape(B, K * E)

    pred = _head(flat, treatment, tr_w, tr_b, cf_b.reshape(1, H),
                 re_w, re_b.reshape(1, H),
                 op_w1, op_b1.reshape(1, H), op_w2, op_b2.reshape(1, H // 2),
                 op_w3, op_b3.reshape(1, OUT))
    return (pred, cemb, scores, indices)


# speedup vs baseline: 2.5567x; 2.5567x over previous
import functools

import jax
import jax.numpy as jnp
from jax.experimental import pallas as pl
from jax.experimental.pallas import tpu as pltpu
from jax.experimental.pallas import tpu_sc as plsc

B = 1024
D_IN = 768
H = 512
E = 768
K = 8
N = 100000
OUT = 128

NB = 1024
NPAD = 100352
NBLK = NPAD // NB
CHUNKS = NB // 128

_NEG_INF = float("-inf")


def _mlp2(x, w1, b1, w2, b2):
    return jax.nn.relu(x @ w1 + b1) @ w2 + b2


def _normalize(x, eps=1e-12):
    n = jnp.linalg.norm(x, axis=1, keepdims=True)
    return x / jnp.maximum(n, eps)


def _insert_candidate(x, xi, v1, v2, v3, v4, c1, c2, c3, c4):
    r1, r2, r3, r4 = v1[...], v2[...], v3[...], v4[...]
    q1, q2, q3, q4 = c1[...], c2[...], c3[...], c4[...]
    g1 = x > r1
    g2 = x > r2
    g3 = x > r3
    g4 = x > r4
    v4[...] = jnp.where(g3, r3, jnp.where(g4, x, r4))
    c4[...] = jnp.where(g3, q3, jnp.where(g4, xi, q4))
    v3[...] = jnp.where(g2, r2, jnp.where(g3, x, r3))
    c3[...] = jnp.where(g2, q2, jnp.where(g3, xi, q3))
    v2[...] = jnp.where(g1, r1, jnp.where(g2, x, r2))
    c2[...] = jnp.where(g1, q1, jnp.where(g2, xi, q2))
    v1[...] = jnp.where(g1, x, r1)
    c1[...] = jnp.where(g1, xi, q1)


def _score_body(qb_ref, corpn_ref, corpraw_ref, nuw1_ref, nub1_ref, nuw2_ref,
                nub2_ref, scores_ref, idx_ref,
                v1, v2, v3, v4, c1, c2, c3, c4):
    i = pl.program_id(0)

    @pl.when(i == 0)
    def _init():
        neg = jnp.full((B, 128), _NEG_INF, jnp.float32)
        zero = jnp.zeros((B, 128), jnp.int32)
        v1[...] = neg
        v2[...] = neg
        v3[...] = neg
        v4[...] = neg
        c1[...] = zero
        c2[...] = zero
        c3[...] = zero
        c4[...] = zero

    h = jax.lax.dot_general(
        corpraw_ref[...].astype(jnp.bfloat16), nuw1_ref[...].astype(jnp.bfloat16),
        (((1,), (0,)), ((), ())), preferred_element_type=jnp.float32)
    h = jnp.maximum(h + nub1_ref[...], 0.0)
    nu2 = jax.lax.dot_general(
        nuw2_ref[...].astype(jnp.bfloat16), h.astype(jnp.bfloat16),
        (((1,), (1,)), ((), ())), preferred_element_type=jnp.float32)
    nu = jax.nn.sigmoid(nu2 + nub2_ref[...])

    sim = jax.lax.dot_general(
        qb_ref[...].astype(jnp.bfloat16), corpn_ref[...].astype(jnp.bfloat16),
        (((1,), (1,)), ((), ())), preferred_element_type=jnp.float32)
    w = sim * nu

    m1 = jnp.full((B, 128), _NEG_INF, jnp.float32)
    m2 = jnp.full((B, 128), _NEG_INF, jnp.float32)
    j1 = jnp.zeros((B, 128), jnp.int32)
    j2 = jnp.zeros((B, 128), jnp.int32)
    lane = jax.lax.broadcasted_iota(jnp.int32, (B, 128), 1)
    for c in range(CHUNKS):
        x = w[:, c * 128:(c + 1) * 128]
        bc = i * CHUNKS + c
        x = jnp.where(bc * 128 + lane < N, x, _NEG_INF)
        g1 = x > m1
        g2 = x > m2
        j2 = jnp.where(g1, j1, jnp.where(g2, bc, j2))
        m2 = jnp.where(g1, m1, jnp.where(g2, x, m2))
        j1 = jnp.where(g1, bc, j1)
        m1 = jnp.where(g1, x, m1)

    _insert_candidate(m1, j1, v1, v2, v3, v4, c1, c2, c3, c4)
    _insert_candidate(m2, j2, v1, v2, v3, v4, c1, c2, c3, c4)

    @pl.when(i == NBLK - 1)
    def _extract():
        cand_v = jnp.concatenate([v1[...], v2[...], v3[...], v4[...]], axis=1)
        cand_b = jnp.concatenate([c1[...], c2[...], c3[...], c4[...]], axis=1)
        lane = jax.lax.broadcasted_iota(jnp.int32, (B, 512), 1) % 128
        gcol = cand_b * 128 + lane
        cand_v = jnp.where(gcol >= N, _NEG_INF, cand_v)
        lane128 = jax.lax.broadcasted_iota(jnp.int32, (B, 128), 1)
        s_out = jnp.zeros((B, 128), jnp.float32)
        i_out = jnp.zeros((B, 128), jnp.int32)
        big = jnp.int32(2**31 - 1)
        for k in range(K):
            vmax = jnp.max(cand_v, axis=1, keepdims=True)
            ism = cand_v == vmax
            selidx = jnp.min(jnp.where(ism, gcol, big), axis=1, keepdims=True)
            s_out = jnp.where(lane128 == k, vmax, s_out)
            i_out = jnp.where(lane128 == k, selidx, i_out)
            cand_v = jnp.where(ism & (gcol == selidx), _NEG_INF, cand_v)
        scores_ref[...] = s_out
        idx_ref[...] = i_out


def _score_topk(qb, corpn_b, corpraw_b, nuw1_b, nub1, nuw2_b, nub2,
                interpret=False):
    scores_p, idx_p = pl.pallas_call(
        _score_body,
        grid=(NBLK,),
        in_specs=[
            pl.BlockSpec((B, E), lambda i: (0, 0)),
            pl.BlockSpec((NB, E), lambda i: (i, 0)),
            pl.BlockSpec((NB, E), lambda i: (i, 0)),
            pl.BlockSpec((E, 128), lambda i: (0, 0)),
            pl.BlockSpec((1, 128), lambda i: (0, 0)),
            pl.BlockSpec((1, 128), lambda i: (0, 0)),
            pl.BlockSpec((1, 1), lambda i: (0, 0)),
        ],
        out_specs=[
            pl.BlockSpec((B, 128), lambda i: (0, 0)),
            pl.BlockSpec((B, 128), lambda i: (0, 0)),
        ],
        out_shape=[
            jax.ShapeDtypeStruct((B, 128), jnp.float32),
            jax.ShapeDtypeStruct((B, 128), jnp.int32),
        ],
        scratch_shapes=[pltpu.VMEM((B, 128), jnp.float32)] * 4
                      + [pltpu.VMEM((B, 128), jnp.int32)] * 4,
        compiler_params=pltpu.CompilerParams(
            dimension_semantics=("arbitrary",)),
        interpret=interpret,
    )(qb, corpn_b, corpraw_b, nuw1_b, nub1, nuw2_b, nub2)
    return scores_p[:, :K], idx_p[:, :K]


_GW = 128
_EH = E // 2


def _sc_gather(corpus_half, indices_flat):
    mesh = plsc.VectorSubcoreMesh(core_axis_name="core",
                                  subcore_axis_name="subcore")
    num_idx = indices_flat.shape[1]

    @functools.partial(
        pl.kernel,
        out_type=jax.ShapeDtypeStruct((num_idx, _EH), jnp.float32),
        mesh=mesh)
    def gk(x_hbm, i_hbm, o_hbm):
        def body(i_vmem, o_vmem):
            pltpu.sync_copy(x_hbm.at[i_vmem.at[0]], o_vmem)

        pltpu.emit_pipeline(
            body,
            grid=(num_idx // _GW,),
            in_specs=[pl.BlockSpec((1, _GW), lambda i: (0, i))],
            out_specs=[pl.BlockSpec((_GW, _EH), lambda i: (i, 0))],
            core_axis_name=("core", "subcore"),
            dimension_semantics=(pltpu.PARALLEL,),
        )(i_hbm, o_hbm)

    return gk(corpus_half, indices_flat)


_HI = jax.lax.Precision.HIGHEST


def _head_body(flat_ref, t_ref, trw_ref, trb_ref, cfb_ref, rew_ref, reb_ref,
               w1_ref, b1_ref, w2_ref, b2_ref, w3_ref, b3_ref, out_ref):
    t_enc = jnp.dot(t_ref[...], trw_ref[...], precision=_HI) + trb_ref[...]
    r_enc = jnp.dot(flat_ref[...], rew_ref[...], precision=_HI) + reb_ref[...]
    cpart = jnp.dot(cfb_ref[...], w1_ref[H:2 * H, :], precision=_HI)
    h1 = jnp.maximum(
        jnp.dot(t_enc, w1_ref[0:H, :], precision=_HI) + cpart
        + jnp.dot(r_enc, w1_ref[2 * H:3 * H, :], precision=_HI)
        + b1_ref[...], 0.0)
    h2 = jnp.maximum(jnp.dot(h1, w2_ref[...], precision=_HI) + b2_ref[...], 0.0)
    out_ref[...] = jnp.dot(h2, w3_ref[...], precision=_HI) + b3_ref[...]


def _head(flat, treatment, tr_w, tr_b, cf_b, re_w, re_b,
          op_w1, op_b1, op_w2, op_b2, op_w3, op_b3, interpret=False):
    RB = 256
    return pl.pallas_call(
        _head_body,
        grid=(B // RB,),
        in_specs=[
            pl.BlockSpec((RB, K * E), lambda i: (i, 0)),
            pl.BlockSpec((RB, D_IN), lambda i: (i, 0)),
            pl.BlockSpec((D_IN, H), lambda i: (0, 0)),
            pl.BlockSpec((1, H), lambda i: (0, 0)),
            pl.BlockSpec((1, H), lambda i: (0, 0)),
            pl.BlockSpec((K * E, H), lambda i: (0, 0)),
            pl.BlockSpec((1, H), lambda i: (0, 0)),
            pl.BlockSpec((3 * H, H), lambda i: (0, 0)),
            pl.BlockSpec((1, H), lambda i: (0, 0)),
            pl.BlockSpec((H, H // 2), lambda i: (0, 0)),
            pl.BlockSpec((1, H // 2), lambda i: (0, 0)),
            pl.BlockSpec((H // 2, OUT), lambda i: (0, 0)),
            pl.BlockSpec((1, OUT), lambda i: (0, 0)),
        ],
        out_specs=pl.BlockSpec((RB, OUT), lambda i: (i, 0)),
        out_shape=jax.ShapeDtypeStruct((B, OUT), jnp.float32),
        compiler_params=pltpu.CompilerParams(
            dimension_semantics=("arbitrary",)),
        interpret=interpret,
    )(flat, treatment, tr_w, tr_b, cf_b, re_w, re_b,
      op_w1, op_b1, op_w2, op_b2, op_w3, op_b3)


def kernel(patient, treatment, confounders, outcome_features, corpus_embeddings,
           pop_w1, pop_b1, pop_w2, pop_b2, int_w1, int_b1, int_w2, int_b2,
           cmp_w1, cmp_b1, cmp_w2, cmp_b2, out_w1, out_b1, out_w2, out_b2,
           eff_w1, eff_b1, eff_w2, eff_b2, nu_w1, nu_b1, nu_w2, nu_b2,
           tr_w, tr_b, cf_w, cf_b, re_w, re_b,
           op_w1, op_b1, op_w2, op_b2, op_w3, op_b3):
    control = jnp.zeros_like(treatment)
    population = _mlp2(patient, pop_w1, pop_b1, pop_w2, pop_b2)
    intervention = _mlp2(treatment, int_w1, int_b1, int_w2, int_b2)
    comparison = _mlp2(control, cmp_w1, cmp_b1, cmp_w2, cmp_b2)
    pi = jnp.concatenate([population, intervention], axis=1)
    pc = jnp.concatenate([population, comparison], axis=1)
    pi_eff = _mlp2(pi, eff_w1, eff_b1, eff_w2, eff_b2)
    pc_eff = _mlp2(pc, eff_w1, eff_b1, eff_w2, eff_b2)
    cemb = _normalize(pi_eff - pc_eff)
    q = _normalize(cemb)
    corp = _normalize(corpus_embeddings)
    q, corp = jax.lax.optimization_barrier((q, corp))

    nub1 = nu_b1.reshape(1, 128)
    nuw2_r = nu_w2.reshape(1, 128)
    nub2 = nu_b2.reshape(1, 1)

    scores, indices = _score_topk(q, corp, corpus_embeddings, nu_w1, nub1,
                                  nuw2_r, nub2)

    idx_flat = indices.reshape(B * K)
    idx_half = jnp.stack([2 * idx_flat, 2 * idx_flat + 1], axis=-1)
    retrieved = _sc_gather(corpus_embeddings.reshape(2 * N, _EH),
                           idx_half.reshape(1, 2 * B * K))
    flat = retrieved.reshape(B, K * E)

    pred = _head(flat, treatment, tr_w, tr_b.reshape(1, H), cf_b.reshape(1, H),
                 re_w, re_b.reshape(1, H),
                 op_w1, op_b1.reshape(1, H), op_w2, op_b2.reshape(1, H // 2),
                 op_w3, op_b3.reshape(1, OUT))
    return (pred, cemb, scores, indices)
